# speedup vs baseline: 2.1188x; 2.1188x over previous
import functools

import jax
import jax.numpy as jnp
from jax import lax
from jax.experimental import pallas as pl
from jax.experimental.pallas import tpu as pltpu
from jax.experimental.pallas import tpu_sc as plsc

_NUM_CORES = 2
_NUM_SUBCORES = 16
_NUM_WORKERS = _NUM_CORES * _NUM_SUBCORES

_CHUNK = 64


@functools.partial(jax.jit, static_argnames=("n_rows", "hidden"))
def _lookup(positions_flat, table, *, n_rows, hidden):
    per_w = n_rows // _NUM_WORKERS
    n_chunks = per_w // _CHUNK
    mesh = plsc.VectorSubcoreMesh(core_axis_name="c", subcore_axis_name="s")

    @functools.partial(
        pl.kernel,
        mesh=mesh,
        out_type=jax.ShapeDtypeStruct((n_rows, hidden), jnp.float32),
        scratch_types=[
            pltpu.VMEM((_CHUNK,), jnp.int32),
            pltpu.VMEM((_CHUNK, hidden), jnp.float32),
            pltpu.SemaphoreType.DMA,
        ],
    )
    def emb_kernel(idx_hbm, table_hbm, out_hbm, idx_v, rows_v, gsem):
        wid = lax.axis_index("s") * _NUM_CORES + lax.axis_index("c")
        base = wid * per_w

        def step(g, carry):
            off = base + g * _CHUNK
            pltpu.sync_copy(idx_hbm.at[pl.ds(off, _CHUNK)], idx_v)
            pltpu.async_copy(table_hbm.at[idx_v], rows_v, gsem).wait()
            pltpu.sync_copy(rows_v, out_hbm.at[pl.ds(off, _CHUNK)])
            return carry

        lax.fori_loop(0, n_chunks, step, 0)

    return emb_kernel(positions_flat, table)


def kernel(positions, pos_embedding):
    b, t = positions.shape
    n_rows = b * t
    hidden = pos_embedding.shape[1]
    flat = positions.reshape(n_rows).astype(jnp.int32)
    out = _lookup(flat, pos_embedding, n_rows=n_rows, hidden=hidden)
    return out.reshape(b, t, hidden)


# speedup vs baseline: 2.3170x; 1.0935x over previous
import functools

import jax
import jax.numpy as jnp
from jax import lax
from jax.experimental import pallas as pl
from jax.experimental.pallas import tpu as pltpu
from jax.experimental.pallas import tpu_sc as plsc

_NUM_CORES = 2
_NUM_SUBCORES = 16
_NUM_WORKERS = _NUM_CORES * _NUM_SUBCORES

_CHUNK = 32


@functools.partial(jax.jit, static_argnames=("n_rows", "hidden"))
def _lookup(positions2d, table, *, n_rows, hidden):
    per_w = n_rows // _NUM_WORKERS
    n_chunks = per_w // _CHUNK
    mesh = plsc.VectorSubcoreMesh(core_axis_name="c", subcore_axis_name="s")

    @functools.partial(
        pl.kernel,
        mesh=mesh,
        out_type=jax.ShapeDtypeStruct((n_rows, hidden), jnp.float32),
        scratch_types=[
            pltpu.VMEM((n_chunks, _CHUNK), jnp.int32),
            pltpu.VMEM((_CHUNK, hidden), jnp.float32),
            pltpu.VMEM((_CHUNK, hidden), jnp.float32),
            pltpu.SemaphoreType.DMA,
            pltpu.SemaphoreType.DMA,
            pltpu.SemaphoreType.DMA,
            pltpu.SemaphoreType.DMA,
        ],
    )
    def emb_kernel(idx_hbm, table_hbm, out_hbm, idx_v, rows0, rows1,
                   gsem0, gsem1, osem0, osem1):
        wid = lax.axis_index("s") * _NUM_CORES + lax.axis_index("c")
        base = wid * per_w
        chunk_row = wid * n_chunks

        pltpu.sync_copy(idx_hbm.at[pl.ds(chunk_row, n_chunks)], idx_v)

        rows = (rows0, rows1)
        gsem = (gsem0, gsem1)
        osem = (osem0, osem1)
        gcp = [None] * n_chunks
        ocp = [None] * n_chunks

        def writeback(g):
            b = g & 1
            gcp[g].wait()
            ocp[g] = pltpu.async_copy(
                rows[b], out_hbm.at[pl.ds(base + g * _CHUNK, _CHUNK)], osem[b])

        for g in range(n_chunks):
            b = g & 1
            if g >= 2:
                ocp[g - 2].wait()
            gcp[g] = pltpu.async_copy(table_hbm.at[idx_v.at[g]], rows[b], gsem[b])
            if g >= 1:
                writeback(g - 1)

        writeback(n_chunks - 1)
        ocp[n_chunks - 2].wait()
        ocp[n_chunks - 1].wait()

    return emb_kernel(positions2d, table)


def kernel(positions, pos_embedding):
    b, t = positions.shape
    n_rows = b * t
    hidden = pos_embedding.shape[1]
    pos2d = positions.reshape(n_rows // _CHUNK, _CHUNK).astype(jnp.int32)
    out = _lookup(pos2d, pos_embedding, n_rows=n_rows, hidden=hidden)
    return out.reshape(b, t, hidden)
